# baseline (speedup 1.0000x reference)
import functools

import jax
import jax.numpy as jnp
from jax import lax
from jax.experimental import pallas as pl
from jax.experimental.pallas import tpu as pltpu
from jax.experimental.pallas import tpu_sc as plsc

N = 10000
E = 320000
D = 128
EPS = 1e-7

NC = 2
NS = 16
NW = NC * NS
RPT = 632
NPAD = NS * RPT

BE = 2000
CHUNK = 40
SPLITS = (125 * NW * CHUNK, 125 * NW * CHUNK)


def _lane_mask(shape):
    return lax.broadcasted_iota(jnp.int32, shape, len(shape) - 1) >= 1


def _arccosh(z):
    return jnp.log(z + jnp.sqrt((z - 1.0) * (z + 1.0)))


def _cosh_sinh(t):
    e = jnp.exp(t)
    ei = 1.0 / e
    return 0.5 * (e + ei), 0.5 * (e - ei)


def _node_linear_body(h_ref, w_ref, b_ref, x_ref):
    h = h_ref[...]
    sp = _lane_mask(h.shape)
    h0 = h[:, 0:1]
    d = _arccosh(jnp.maximum(h0, 1.0 + EPS))
    n = jnp.maximum(
        jnp.sqrt(jnp.sum(jnp.where(sp, h * h, 0.0), axis=1, keepdims=True)), 1e-8)
    lm = jnp.where(sp, h * (d / n), 0.0)
    xt = jnp.dot(lm, w_ref[...], preferred_element_type=jnp.float32)
    n2 = jnp.maximum(
        jnp.sqrt(jnp.sum(jnp.where(sp, xt * xt, 0.0), axis=1, keepdims=True)), 1e-8)
    c2, s2 = _cosh_sinh(n2)
    x = jnp.where(sp, xt * (s2 / n2), c2)
    bf = b_ref[...]
    ip = jnp.sum(jnp.where(sp, x * bf, 0.0), axis=1, keepdims=True)
    coef = ip / (1.0 + x[:, 0:1])
    bias = jnp.where(sp, bf + coef * x, ip)
    linn = jnp.sum(jnp.where(sp, bias * bias, -(bias * bias)), axis=1, keepdims=True)
    nrm = jnp.sqrt(jnp.maximum(linn, 1e-12))
    c3, s3 = _cosh_sinh(nrm)
    x_ref[...] = c3 * x + (s3 / nrm) * bias


def _stage_a(h, w, b):
    return pl.pallas_call(
        _node_linear_body,
        out_shape=jax.ShapeDtypeStruct((N, D), jnp.float32),
    )(h, w, b)


def _sc_gather(x, row3, col3):
    nchunk = row3.shape[1]
    ept = nchunk * CHUNK
    e_tot = NW * ept

    def body(x_hbm, row_hbm, col_hbm, xr_hbm, xc_hbm,
             ridx_all, cidx_all, rbuf_a, cbuf_a, rbuf_b, cbuf_b,
             sem_a, sem_b):
        wid = lax.axis_index("s") * NC + lax.axis_index("c")
        base = wid * ept
        pltpu.sync_copy(row_hbm.at[wid], ridx_all)
        pltpu.sync_copy(col_hbm.at[wid], cidx_all)

        def fire(c, rbuf, cbuf, sem):
            pltpu.async_copy(x_hbm.at[ridx_all.at[c]], rbuf, sem)
            pltpu.async_copy(x_hbm.at[cidx_all.at[c]], cbuf, sem)

        def drain_store(c, rbuf, cbuf, sem):
            pltpu.make_async_copy(x_hbm.at[ridx_all.at[c]], rbuf, sem).wait()
            pltpu.make_async_copy(x_hbm.at[cidx_all.at[c]], cbuf, sem).wait()
            off = base + c * CHUNK
            pltpu.sync_copy(rbuf, xr_hbm.at[pl.ds(off, CHUNK)])
            pltpu.sync_copy(cbuf, xc_hbm.at[pl.ds(off, CHUNK)])

        fire(0, rbuf_a, cbuf_a, sem_a)

        def loop(p, carry):
            c = 2 * p
            fire(c + 1, rbuf_b, cbuf_b, sem_b)
            drain_store(c, rbuf_a, cbuf_a, sem_a)
            fire(c + 2, rbuf_a, cbuf_a, sem_a)
            drain_store(c + 1, rbuf_b, cbuf_b, sem_b)
            return carry

        lax.fori_loop(0, (nchunk - 1) // 2, loop, 0)
        drain_store(nchunk - 1, rbuf_a, cbuf_a, sem_a)

    fn = functools.partial(
        pl.kernel,
        out_type=[jax.ShapeDtypeStruct((e_tot, D), jnp.float32),
                  jax.ShapeDtypeStruct((e_tot, D), jnp.float32)],
        mesh=plsc.VectorSubcoreMesh(core_axis_name="c", subcore_axis_name="s"),
        scratch_types=[
            pltpu.VMEM((nchunk, CHUNK), jnp.int32),
            pltpu.VMEM((nchunk, CHUNK), jnp.int32),
            pltpu.VMEM((CHUNK, D), jnp.float32),
            pltpu.VMEM((CHUNK, D), jnp.float32),
            pltpu.VMEM((CHUNK, D), jnp.float32),
            pltpu.VMEM((CHUNK, D), jnp.float32),
            pltpu.SemaphoreType.DMA,
            pltpu.SemaphoreType.DMA,
        ],
    )(body)
    return fn(x, row3, col3)


def _edge_body(xr_ref, xc_ref, ea_ref, em_ref,
               w1r_ref, w1c_ref, w1e_ref, b1_ref, w2_ref, b2_ref, agg_ref):
    gr = xr_ref[...]
    gc = xc_ref[...]

    def tan_scale(g0):
        g0c = jnp.maximum(g0, 1.0 + EPS)
        n = jnp.sqrt((g0c - 1.0) * (g0c + 1.0))
        return jnp.log(g0c + n) / jnp.maximum(n, 1e-8)

    tr = gr * tan_scale(gr[:, 0:1])
    tcv = gc * tan_scale(gc[:, 0:1])
    s = jnp.dot(tr, w1r_ref[...], preferred_element_type=jnp.float32)
    s = s + jnp.dot(tcv, w1c_ref[...], preferred_element_type=jnp.float32)
    s = s + jnp.dot(ea_ref[...], w1e_ref[...], preferred_element_type=jnp.float32)
    s = s + b1_ref[...]
    hid = s / (1.0 + jnp.exp(-s))
    logit = jnp.sum(hid * w2_ref[...], axis=1, keepdims=True) + b2_ref[...][:, 0:1]
    att = em_ref[...] / (1.0 + jnp.exp(-logit))
    alpha = jnp.maximum(
        2.0 * gr[:, 0:1] * gc[:, 0:1] - jnp.sum(gr * gc, axis=1, keepdims=True),
        1.0 + EPS)
    ssq = jnp.sqrt((alpha - 1.0) * (alpha + 1.0))
    dd = jnp.log(alpha + ssq)
    agg_ref[...] = (att * dd / ssq) * (gc - alpha * gr)


def _edge_stage(xr, xc, ea_pad, edge_mask, w1r, w1c, w1e, b1, w2row, b2row):
    e_tot = xr.shape[0]
    grid = (e_tot // BE,)
    return pl.pallas_call(
        _edge_body,
        grid=grid,
        in_specs=[
            pl.BlockSpec((BE, D), lambda i: (i, 0)),
            pl.BlockSpec((BE, D), lambda i: (i, 0)),
            pl.BlockSpec((BE, 8), lambda i: (i, 0)),
            pl.BlockSpec((BE, 1), lambda i: (i, 0)),
            pl.BlockSpec((D, D), lambda i: (0, 0)),
            pl.BlockSpec((D, D), lambda i: (0, 0)),
            pl.BlockSpec((8, D), lambda i: (0, 0)),
            pl.BlockSpec((1, D), lambda i: (0, 0)),
            pl.BlockSpec((1, D), lambda i: (0, 0)),
            pl.BlockSpec((1, D), lambda i: (0, 0)),
        ],
        out_specs=pl.BlockSpec((BE, D), lambda i: (i, 0)),
        out_shape=jax.ShapeDtypeStruct((e_tot, D), jnp.float32),
    )(xr, xc, ea_pad, edge_mask, w1r, w1c, w1e, b1, w2row, b2row)


def _sc_scatter(agg, row3, zeros_tile):
    nchunk = row3.shape[1]
    ept = nchunk * CHUNK

    def body(agg_hbm, row_hbm, z_hbm, out_hbm,
             idx_all, val_a, val_b, acc_sh, sem_a, sem_b):
        c = lax.axis_index("c")
        s = lax.axis_index("s")
        wid = s * NC + c
        rbase = s * RPT
        pltpu.sync_copy(z_hbm, acc_sh.at[pl.ds(rbase, RPT)])
        pltpu.sync_copy(row_hbm.at[wid], idx_all)
        plsc.subcore_barrier()

        def fire(j, val, sem):
            off = wid * ept + j * CHUNK
            pltpu.async_copy(agg_hbm.at[pl.ds(off, CHUNK)], val, sem)

        def drain_scatter(j, val, sem):
            off = wid * ept + j * CHUNK
            pltpu.make_async_copy(agg_hbm.at[pl.ds(off, CHUNK)], val, sem).wait()
            pltpu.sync_copy(val, acc_sh.at[idx_all.at[j]], add=True)

        fire(0, val_a, sem_a)

        def loop(p, carry):
            j = 2 * p
            fire(j + 1, val_b, sem_b)
            drain_scatter(j, val_a, sem_a)
            fire(j + 2, val_a, sem_a)
            drain_scatter(j + 1, val_b, sem_b)
            return carry

        lax.fori_loop(0, (nchunk - 1) // 2, loop, 0)
        drain_scatter(nchunk - 1, val_a, sem_a)
        plsc.subcore_barrier()
        pltpu.sync_copy(acc_sh.at[pl.ds(rbase, RPT)],
                        out_hbm.at[c, pl.ds(rbase, RPT)])

    fn = functools.partial(
        pl.kernel,
        out_type=jax.ShapeDtypeStruct((NC, NPAD, D), jnp.float32),
        mesh=plsc.VectorSubcoreMesh(core_axis_name="c", subcore_axis_name="s"),
        scratch_types=[
            pltpu.VMEM((nchunk, CHUNK), jnp.int32),
            pltpu.VMEM((CHUNK, D), jnp.float32),
            pltpu.VMEM((CHUNK, D), jnp.float32),
            pltpu.VMEM_SHARED((NPAD, D), jnp.float32),
            pltpu.SemaphoreType.DMA,
            pltpu.SemaphoreType.DMA,
        ],
    )(body)
    return fn(agg, row3, zeros_tile)


def _final_body(x_ref, *refs):
    p_refs = refs[:-3]
    gam_ref, bet_ref, out_ref = refs[-3:]
    x = x_ref[...]
    sp = _lane_mask(x.shape)
    agg = p_refs[0][...]
    for p_ref in p_refs[1:]:
        agg = agg + p_ref[...]
    li = jnp.sum(jnp.where(sp, x * agg, -(x * agg)), axis=1, keepdims=True)
    u = agg + li * x
    linn = jnp.sum(jnp.where(sp, u * u, -(u * u)), axis=1, keepdims=True)
    nrm = jnp.sqrt(jnp.maximum(linn, 1e-12))
    ch, sh = _cosh_sinh(nrm)
    x2 = ch * x + (sh / nrm) * u
    d = _arccosh(jnp.maximum(x2[:, 0:1], 1.0 + EPS))
    n = jnp.maximum(
        jnp.sqrt(jnp.sum(jnp.where(sp, x2 * x2, 0.0), axis=1, keepdims=True)), 1e-8)
    ht = jnp.where(sp, x2 * (d / n), 0.0)
    mu = jnp.sum(ht, axis=1, keepdims=True) / 127.0
    dsp = jnp.where(sp, ht - mu, 0.0)
    var = jnp.sum(dsp * dsp, axis=1, keepdims=True) / 127.0
    spn = dsp / jnp.sqrt(var + 1e-5) * gam_ref[...] + bet_ref[...]
    n3 = jnp.maximum(
        jnp.sqrt(jnp.sum(jnp.where(sp, spn * spn, 0.0), axis=1, keepdims=True)),
        1e-8)
    c3, s3 = _cosh_sinh(n3)
    x3 = jnp.where(sp, spn * (s3 / n3), c3)
    d4 = _arccosh(jnp.maximum(x3[:, 0:1], 1.0 + EPS))
    n4 = jnp.maximum(
        jnp.sqrt(jnp.sum(jnp.where(sp, x3 * x3, 0.0), axis=1, keepdims=True)), 1e-8)
    r = jnp.maximum(jnp.where(sp, x3 * (d4 / n4), 0.0), 0.0)
    n5 = jnp.maximum(jnp.sqrt(jnp.sum(r * r, axis=1, keepdims=True)), 1e-8)
    c5, s5 = _cosh_sinh(n5)
    out_ref[...] = jnp.where(sp, r * (s5 / n5), c5)


def _final_stage(x, partials, gam, bet):
    return pl.pallas_call(
        _final_body,
        out_shape=jax.ShapeDtypeStruct((N, D), jnp.float32),
    )(x, *partials, gam, bet)


def kernel(h, edge_attr, edges, node_mask, edge_mask, W, b, gamma, beta,
           aW1, ab1, aW2, ab2):
    del node_mask
    row = edges[0].astype(jnp.int32)
    col = edges[1].astype(jnp.int32)

    x = _stage_a(h, W, b)

    ea_pad = jnp.concatenate(
        [edge_attr, jnp.zeros((E, 8 - edge_attr.shape[1]), jnp.float32)], axis=1)
    lane0 = jnp.arange(D)[:, None] > 0
    w1r = aW1[:D] * lane0
    w1c = aW1[D:2 * D] * lane0
    w1e = jnp.concatenate(
        [aW1[2 * D:], jnp.zeros((8 - (aW1.shape[0] - 2 * D), D), jnp.float32)],
        axis=0)
    b1 = ab1.reshape(1, D)
    w2row = aW2.reshape(1, D)
    b2row = jnp.broadcast_to(ab2.reshape(1, 1), (1, D))
    zeros_tile = jnp.zeros((RPT, D), jnp.float32)

    partials = []
    start = 0
    for eh in SPLITS:
        sl = slice(start, start + eh)
        start += eh
        nch = eh // (NW * CHUNK)
        r3 = row[sl].reshape(NW, nch, CHUNK)
        c3 = col[sl].reshape(NW, nch, CHUNK)
        xr, xc = _sc_gather(x, r3, c3)
        agg = _edge_stage(xr, xc, ea_pad[sl], edge_mask[sl], w1r, w1c, w1e,
                          b1, w2row, b2row)
        part = _sc_scatter(agg, r3, zeros_tile)
        partials.extend([part[0, :N], part[1, :N]])

    gam = jnp.concatenate([jnp.ones((1,), jnp.float32), gamma]).reshape(1, D)
    bet = jnp.concatenate([jnp.zeros((1,), jnp.float32), beta]).reshape(1, D)
    return _final_stage(x, partials, gam, bet)


# speedup vs baseline: 1.0609x; 1.0609x over previous
import functools

import jax
import jax.numpy as jnp
from jax import lax
from jax.experimental import pallas as pl
from jax.experimental.pallas import tpu as pltpu
from jax.experimental.pallas import tpu_sc as plsc

N = 10000
E = 320000
D = 128
EPS = 1e-7

NC = 2
NS = 16
NW = NC * NS
RPT = 632
NPAD = NS * RPT

BE = 2000
CHUNK = 40
SPLITS = (125 * NW * CHUNK, 125 * NW * CHUNK)


def _lane_mask(shape):
    return lax.broadcasted_iota(jnp.int32, shape, len(shape) - 1) >= 1


def _arccosh(z):
    return jnp.log(z + jnp.sqrt((z - 1.0) * (z + 1.0)))


def _cosh_sinh(t):
    e = jnp.exp(t)
    ei = 1.0 / e
    return 0.5 * (e + ei), 0.5 * (e - ei)


def _node_linear_body(h_ref, w_ref, b_ref, x_ref):
    h = h_ref[...]
    sp = _lane_mask(h.shape)
    h0 = h[:, 0:1]
    d = _arccosh(jnp.maximum(h0, 1.0 + EPS))
    n = jnp.maximum(
        jnp.sqrt(jnp.sum(jnp.where(sp, h * h, 0.0), axis=1, keepdims=True)), 1e-8)
    lm = jnp.where(sp, h * (d / n), 0.0)
    xt = jnp.dot(lm, w_ref[...], preferred_element_type=jnp.float32)
    n2 = jnp.maximum(
        jnp.sqrt(jnp.sum(jnp.where(sp, xt * xt, 0.0), axis=1, keepdims=True)), 1e-8)
    c2, s2 = _cosh_sinh(n2)
    x = jnp.where(sp, xt * (s2 / n2), c2)
    bf = b_ref[...]
    ip = jnp.sum(jnp.where(sp, x * bf, 0.0), axis=1, keepdims=True)
    coef = ip / (1.0 + x[:, 0:1])
    bias = jnp.where(sp, bf + coef * x, ip)
    linn = jnp.sum(jnp.where(sp, bias * bias, -(bias * bias)), axis=1, keepdims=True)
    nrm = jnp.sqrt(jnp.maximum(linn, 1e-12))
    c3, s3 = _cosh_sinh(nrm)
    x_ref[...] = c3 * x + (s3 / nrm) * bias


def _stage_a(h, w, b):
    return pl.pallas_call(
        _node_linear_body,
        out_shape=jax.ShapeDtypeStruct((N, D), jnp.float32),
    )(h, w, b)


def _sc_gather(x, row1, col1):
    e_tot = row1.shape[0]
    ept = e_tot // NW
    nchunk = ept // CHUNK

    def body(x_hbm, row_hbm, col_hbm, xr_hbm, xc_hbm,
             ridx_a, cidx_a, rbuf_a, cbuf_a,
             ridx_b, cidx_b, rbuf_b, cbuf_b, sem_a, sem_b):
        wid = lax.axis_index("s") * NC + lax.axis_index("c")
        base = wid * ept

        def issue(c, ridx, cidx, rbuf, cbuf, sem):
            off = base + c * CHUNK
            pltpu.sync_copy(row_hbm.at[pl.ds(off, CHUNK)], ridx)
            pltpu.sync_copy(col_hbm.at[pl.ds(off, CHUNK)], cidx)
            pltpu.async_copy(x_hbm.at[ridx], rbuf, sem)
            pltpu.async_copy(x_hbm.at[cidx], cbuf, sem)

        def drain_store(c, ridx, cidx, rbuf, cbuf, sem):
            pltpu.make_async_copy(x_hbm.at[ridx], rbuf, sem).wait()
            pltpu.make_async_copy(x_hbm.at[cidx], cbuf, sem).wait()
            off = base + c * CHUNK
            pltpu.sync_copy(rbuf, xr_hbm.at[pl.ds(off, CHUNK)])
            pltpu.sync_copy(cbuf, xc_hbm.at[pl.ds(off, CHUNK)])

        issue(0, ridx_a, cidx_a, rbuf_a, cbuf_a, sem_a)

        def loop(p, carry):
            c = 2 * p
            issue(c + 1, ridx_b, cidx_b, rbuf_b, cbuf_b, sem_b)
            drain_store(c, ridx_a, cidx_a, rbuf_a, cbuf_a, sem_a)
            issue(c + 2, ridx_a, cidx_a, rbuf_a, cbuf_a, sem_a)
            drain_store(c + 1, ridx_b, cidx_b, rbuf_b, cbuf_b, sem_b)
            return carry

        lax.fori_loop(0, (nchunk - 1) // 2, loop, 0)
        drain_store(nchunk - 1, ridx_a, cidx_a, rbuf_a, cbuf_a, sem_a)

    fn = functools.partial(
        pl.kernel,
        out_type=[jax.ShapeDtypeStruct((e_tot, D), jnp.float32),
                  jax.ShapeDtypeStruct((e_tot, D), jnp.float32)],
        mesh=plsc.VectorSubcoreMesh(core_axis_name="c", subcore_axis_name="s"),
        scratch_types=[
            pltpu.VMEM((CHUNK,), jnp.int32),
            pltpu.VMEM((CHUNK,), jnp.int32),
            pltpu.VMEM((CHUNK, D), jnp.float32),
            pltpu.VMEM((CHUNK, D), jnp.float32),
            pltpu.VMEM((CHUNK,), jnp.int32),
            pltpu.VMEM((CHUNK,), jnp.int32),
            pltpu.VMEM((CHUNK, D), jnp.float32),
            pltpu.VMEM((CHUNK, D), jnp.float32),
            pltpu.SemaphoreType.DMA,
            pltpu.SemaphoreType.DMA,
        ],
    )(body)
    return fn(x, row1, col1)


def _edge_body(xr_ref, xc_ref, ea_ref, em_ref,
               w1r_ref, w1c_ref, w1e_ref, b1_ref, w2_ref, b2_ref, agg_ref):
    gr = xr_ref[...]
    gc = xc_ref[...]

    def tan_scale(g0):
        g0c = jnp.maximum(g0, 1.0 + EPS)
        n = jnp.sqrt((g0c - 1.0) * (g0c + 1.0))
        return jnp.log(g0c + n) / jnp.maximum(n, 1e-8)

    tr = gr * tan_scale(gr[:, 0:1])
    tcv = gc * tan_scale(gc[:, 0:1])
    s = jnp.dot(tr, w1r_ref[...], preferred_element_type=jnp.float32)
    s = s + jnp.dot(tcv, w1c_ref[...], preferred_element_type=jnp.float32)
    s = s + jnp.dot(ea_ref[...], w1e_ref[...], preferred_element_type=jnp.float32)
    s = s + b1_ref[...]
    hid = s / (1.0 + jnp.exp(-s))
    logit = jnp.sum(hid * w2_ref[...], axis=1, keepdims=True) + b2_ref[...][:, 0:1]
    att = em_ref[...] / (1.0 + jnp.exp(-logit))
    alpha = jnp.maximum(
        2.0 * gr[:, 0:1] * gc[:, 0:1] - jnp.sum(gr * gc, axis=1, keepdims=True),
        1.0 + EPS)
    ssq = jnp.sqrt((alpha - 1.0) * (alpha + 1.0))
    dd = jnp.log(alpha + ssq)
    agg_ref[...] = (att * dd / ssq) * (gc - alpha * gr)


def _edge_stage(xr, xc, ea_pad, edge_mask, w1r, w1c, w1e, b1, w2row, b2row):
    e_tot = xr.shape[0]
    grid = (e_tot // BE,)
    return pl.pallas_call(
        _edge_body,
        grid=grid,
        in_specs=[
            pl.BlockSpec((BE, D), lambda i: (i, 0)),
            pl.BlockSpec((BE, D), lambda i: (i, 0)),
            pl.BlockSpec((BE, 8), lambda i: (i, 0)),
            pl.BlockSpec((BE, 1), lambda i: (i, 0)),
            pl.BlockSpec((D, D), lambda i: (0, 0)),
            pl.BlockSpec((D, D), lambda i: (0, 0)),
            pl.BlockSpec((8, D), lambda i: (0, 0)),
            pl.BlockSpec((1, D), lambda i: (0, 0)),
            pl.BlockSpec((1, D), lambda i: (0, 0)),
            pl.BlockSpec((1, D), lambda i: (0, 0)),
        ],
        out_specs=pl.BlockSpec((BE, D), lambda i: (i, 0)),
        out_shape=jax.ShapeDtypeStruct((e_tot, D), jnp.float32),
    )(xr, xc, ea_pad, edge_mask, w1r, w1c, w1e, b1, w2row, b2row)


def _sc_scatter(agg, row1, zeros_tile):
    e_tot = row1.shape[0]
    ept = e_tot // NW
    nchunk = ept // CHUNK

    def body(agg_hbm, row_hbm, z_hbm, out_hbm,
             idx_a, val_a, idx_b, val_b, acc_sh, sem_a, sem_b):
        c = lax.axis_index("c")
        s = lax.axis_index("s")
        wid = s * NC + c
        rbase = s * RPT
        pltpu.sync_copy(z_hbm, acc_sh.at[pl.ds(rbase, RPT)])
        plsc.subcore_barrier()

        def issue(j, idx, val, sem):
            off = wid * ept + j * CHUNK
            pltpu.async_copy(row_hbm.at[pl.ds(off, CHUNK)], idx, sem)
            pltpu.async_copy(agg_hbm.at[pl.ds(off, CHUNK)], val, sem)

        def drain_scatter(j, idx, val, sem):
            off = wid * ept + j * CHUNK
            pltpu.make_async_copy(row_hbm.at[pl.ds(off, CHUNK)], idx, sem).wait()
            pltpu.make_async_copy(agg_hbm.at[pl.ds(off, CHUNK)], val, sem).wait()
            pltpu.sync_copy(val, acc_sh.at[idx], add=True)

        issue(0, idx_a, val_a, sem_a)

        def loop(p, carry):
            j = 2 * p
            issue(j + 1, idx_b, val_b, sem_b)
            drain_scatter(j, idx_a, val_a, sem_a)
            issue(j + 2, idx_a, val_a, sem_a)
            drain_scatter(j + 1, idx_b, val_b, sem_b)
            return carry

        lax.fori_loop(0, (nchunk - 1) // 2, loop, 0)
        drain_scatter(nchunk - 1, idx_a, val_a, sem_a)
        plsc.subcore_barrier()
        pltpu.sync_copy(acc_sh.at[pl.ds(rbase, RPT)],
                        out_hbm.at[c, pl.ds(rbase, RPT)])

    fn = functools.partial(
        pl.kernel,
        out_type=jax.ShapeDtypeStruct((NC, NPAD, D), jnp.float32),
        mesh=plsc.VectorSubcoreMesh(core_axis_name="c", subcore_axis_name="s"),
        scratch_types=[
            pltpu.VMEM((CHUNK,), jnp.int32),
            pltpu.VMEM((CHUNK, D), jnp.float32),
            pltpu.VMEM((CHUNK,), jnp.int32),
            pltpu.VMEM((CHUNK, D), jnp.float32),
            pltpu.VMEM_SHARED((NPAD, D), jnp.float32),
            pltpu.SemaphoreType.DMA,
            pltpu.SemaphoreType.DMA,
        ],
    )(body)
    return fn(agg, row1, zeros_tile)


def _final_body(x_ref, *refs):
    p_refs = refs[:-3]
    gam_ref, bet_ref, out_ref = refs[-3:]
    x = x_ref[...]
    sp = _lane_mask(x.shape)
    agg = p_refs[0][...]
    for p_ref in p_refs[1:]:
        agg = agg + p_ref[...]
    li = jnp.sum(jnp.where(sp, x * agg, -(x * agg)), axis=1, keepdims=True)
    u = agg + li * x
    linn = jnp.sum(jnp.where(sp, u * u, -(u * u)), axis=1, keepdims=True)
    nrm = jnp.sqrt(jnp.maximum(linn, 1e-12))
    ch, sh = _cosh_sinh(nrm)
    x2 = ch * x + (sh / nrm) * u
    d = _arccosh(jnp.maximum(x2[:, 0:1], 1.0 + EPS))
    n = jnp.maximum(
        jnp.sqrt(jnp.sum(jnp.where(sp, x2 * x2, 0.0), axis=1, keepdims=True)), 1e-8)
    ht = jnp.where(sp, x2 * (d / n), 0.0)
    mu = jnp.sum(ht, axis=1, keepdims=True) / 127.0
    dsp = jnp.where(sp, ht - mu, 0.0)
    var = jnp.sum(dsp * dsp, axis=1, keepdims=True) / 127.0
    spn = dsp / jnp.sqrt(var + 1e-5) * gam_ref[...] + bet_ref[...]
    n3 = jnp.maximum(
        jnp.sqrt(jnp.sum(jnp.where(sp, spn * spn, 0.0), axis=1, keepdims=True)),
        1e-8)
    c3, s3 = _cosh_sinh(n3)
    x3 = jnp.where(sp, spn * (s3 / n3), c3)
    d4 = _arccosh(jnp.maximum(x3[:, 0:1], 1.0 + EPS))
    n4 = jnp.maximum(
        jnp.sqrt(jnp.sum(jnp.where(sp, x3 * x3, 0.0), axis=1, keepdims=True)), 1e-8)
    r = jnp.maximum(jnp.where(sp, x3 * (d4 / n4), 0.0), 0.0)
    n5 = jnp.maximum(jnp.sqrt(jnp.sum(r * r, axis=1, keepdims=True)), 1e-8)
    c5, s5 = _cosh_sinh(n5)
    out_ref[...] = jnp.where(sp, r * (s5 / n5), c5)


def _final_stage(x, partials, gam, bet):
    return pl.pallas_call(
        _final_body,
        out_shape=jax.ShapeDtypeStruct((N, D), jnp.float32),
    )(x, *partials, gam, bet)


def kernel(h, edge_attr, edges, node_mask, edge_mask, W, b, gamma, beta,
           aW1, ab1, aW2, ab2):
    del node_mask
    row = edges[0].astype(jnp.int32)
    col = edges[1].astype(jnp.int32)

    x = _stage_a(h, W, b)

    ea_pad = jnp.concatenate(
        [edge_attr, jnp.zeros((E, 8 - edge_attr.shape[1]), jnp.float32)], axis=1)
    lane0 = jnp.arange(D)[:, None] > 0
    w1r = aW1[:D] * lane0
    w1c = aW1[D:2 * D] * lane0
    w1e = jnp.concatenate(
        [aW1[2 * D:], jnp.zeros((8 - (aW1.shape[0] - 2 * D), D), jnp.float32)],
        axis=0)
    b1 = ab1.reshape(1, D)
    w2row = aW2.reshape(1, D)
    b2row = jnp.broadcast_to(ab2.reshape(1, 1), (1, D))
    zeros_tile = jnp.zeros((RPT, D), jnp.float32)

    partials = []
    start = 0
    for eh in SPLITS:
        sl = slice(start, start + eh)
        start += eh
        r_h, c_h = row[sl], col[sl]
        xr, xc = _sc_gather(x, r_h, c_h)
        agg = _edge_stage(xr, xc, ea_pad[sl], edge_mask[sl], w1r, w1c, w1e,
                          b1, w2row, b2row)
        part = _sc_scatter(agg, r_h, zeros_tile)
        partials.extend([part[0, :N], part[1, :N]])

    gam = jnp.concatenate([jnp.ones((1,), jnp.float32), gamma]).reshape(1, D)
    bet = jnp.concatenate([jnp.zeros((1,), jnp.float32), beta]).reshape(1, D)
    return _final_stage(x, partials, gam, bet)
